# baseline (device time: 23387 ns/iter reference)
import jax
import jax.numpy as jnp
from jax import lax
from jax.experimental import pallas as pl
from jax.experimental.pallas import tpu as pltpu

N_DEV = 4
B, H, D = 8, 8, 64
BH = B * H
HD = H * D
SCALE = D ** -0.5
BC = 2
NC = B // BC
CW = 128


def kernel(Q, K, V):
    Kl = K.shape[1]
    q = Q[:, 0]
    eye8 = jnp.eye(H, dtype=Q.dtype)
    qblk = (q[:, :, None, :] * eye8[None, :, :, None]).reshape(B, H, HD)
    K2 = K.reshape(B, Kl, HD)
    V2 = V.reshape(B, Kl, HD)

    def body(qblk_ref, k_ref, v_ref, out_ref,
             kbuf, vbuf, mine_ref, comm_ref,
             copy_sems, send_sems, recv_sems):
        my_pos = lax.axis_index("i")

        kcopies, vcopies = [], []
        for c in range(NC):
            kc = pltpu.make_async_copy(
                k_ref.at[pl.ds(c * BC, BC)],
                kbuf.at[pl.ds(c * BC, BC)],
                copy_sems.at[2 * c],
            )
            kc.start()
            vc = pltpu.make_async_copy(
                v_ref.at[pl.ds(c * BC, BC)],
                vbuf.at[pl.ds(c * BC, BC)],
                copy_sems.at[2 * c + 1],
            )
            vc.start()
            kcopies.append(kc)
            vcopies.append(vc)

        barrier_sem = pltpu.get_barrier_semaphore()
        for j in range(1, N_DEV):
            pl.semaphore_signal(
                barrier_sem, inc=1,
                device_id=((my_pos + j) % N_DEV,),
                device_id_type=pl.DeviceIdType.MESH,
            )
        pl.semaphore_wait(barrier_sem, N_DEV - 1)

        for c in range(NC):
            kcopies[c].wait()
            ps = []
            for b in range(c * BC, (c + 1) * BC):
                qbT = qblk_ref[b]
                kb = kbuf[b]
                s = lax.dot_general(
                    qbT, kb,
                    dimension_numbers=(((1,), (1,)), ((), ())),
                    preferred_element_type=jnp.float32,
                ) * SCALE
                m = jnp.max(s, axis=1, keepdims=True)
                p = jnp.exp(s - m)
                l = jnp.sum(p, axis=1, keepdims=True)
                mine_ref[pl.ds(b * H, H), D:D + 1] = m
                mine_ref[pl.ds(b * H, H), D + 1:D + 2] = l
                ps.append(p)
            vcopies[c].wait()
            for i, b in enumerate(range(c * BC, (c + 1) * BC)):
                vb = vbuf[b]
                of = lax.dot_general(
                    ps[i], vb,
                    dimension_numbers=(((1,), (0,)), ((), ())),
                    preferred_element_type=jnp.float32,
                )
                hh = lax.broadcasted_iota(jnp.int32, (H, HD), 0)
                blk = lax.broadcasted_iota(jnp.int32, (H, HD), 1) // D
                ofm = jnp.where(hh == blk, of, 0.0)
                ob = ofm[:, 0:D]
                for h in range(1, H):
                    ob = ob + ofm[:, h * D:(h + 1) * D]
                mine_ref[pl.ds(b * H, H), 0:D] = ob

        rdmas = []
        for j in range(1, N_DEV):
            slot = N_DEV - 1 - j
            rdma = pltpu.make_async_remote_copy(
                src_ref=mine_ref,
                dst_ref=comm_ref.at[slot],
                send_sem=send_sems.at[j - 1],
                recv_sem=recv_sems.at[slot],
                device_id=((my_pos + j) % N_DEV,),
                device_id_type=pl.DeviceIdType.MESH,
            )
            rdma.start()
            rdmas.append(rdma)
        for rdma in rdmas:
            rdma.wait()

        m_parts = [mine_ref[:, D:D + 1]] + [
            comm_ref[i, :, D:D + 1] for i in range(N_DEV - 1)
        ]
        l_parts = [mine_ref[:, D + 1:D + 2]] + [
            comm_ref[i, :, D + 1:D + 2] for i in range(N_DEV - 1)
        ]
        m_g = m_parts[0]
        for i in range(1, N_DEV):
            m_g = jnp.maximum(m_g, m_parts[i])
        alphas = [jnp.exp(mp - m_g) for mp in m_parts]
        l_g = alphas[0] * l_parts[0]
        for i in range(1, N_DEV):
            l_g = l_g + alphas[i] * l_parts[i]
        o_acc = alphas[0] * mine_ref[:, 0:D]
        for i in range(1, N_DEV):
            o_acc = o_acc + alphas[i] * comm_ref[i - 1, :, 0:D]
        out_ref[:, :] = o_acc / l_g

    out2 = pl.pallas_call(
        body,
        out_shape=jax.ShapeDtypeStruct((BH, D), jnp.float32),
        in_specs=[
            pl.BlockSpec(memory_space=pltpu.VMEM),
            pl.BlockSpec(memory_space=pl.ANY),
            pl.BlockSpec(memory_space=pl.ANY),
        ],
        out_specs=pl.BlockSpec(memory_space=pltpu.VMEM),
        scratch_shapes=[
            pltpu.VMEM((B, Kl, HD), jnp.float32),
            pltpu.VMEM((B, Kl, HD), jnp.float32),
            pltpu.VMEM((BH, CW), jnp.float32),
            pltpu.VMEM((N_DEV - 1, BH, CW), jnp.float32),
            pltpu.SemaphoreType.DMA((2 * NC,)),
            pltpu.SemaphoreType.DMA((N_DEV - 1,)),
            pltpu.SemaphoreType.DMA((N_DEV - 1,)),
        ],
        compiler_params=pltpu.CompilerParams(collective_id=0),
    )(qblk, K2, V2)
    return out2.reshape(B, 1, H, D)


# device time: 22093 ns/iter; 1.0586x vs baseline; 1.0586x over previous
import jax
import jax.numpy as jnp
from jax import lax
from jax.experimental import pallas as pl
from jax.experimental.pallas import tpu as pltpu

N_DEV = 4
B, H, D = 8, 8, 64
BH = B * H
HD = H * D
SCALE = D ** -0.5
BC = 2
NC = B // BC
CW = 128


def kernel(Q, K, V):
    Kl = K.shape[1]
    q = Q[:, 0]
    eye8 = jnp.eye(H, dtype=Q.dtype)
    qblk = (q[:, :, None, :] * eye8[None, :, :, None]).reshape(B, H, HD)
    K2 = K.reshape(B, Kl, HD)
    V2 = V.reshape(B, Kl, HD)

    def body(qblk_ref, k_ref, v_ref, out_ref,
             kbuf, vbuf, mine_ref, comm_ref,
             copy_sems, send_sems, recv_sems):
        my_pos = lax.axis_index("i")

        kcopies, vcopies = [], []
        for c in range(NC):
            kc = pltpu.make_async_copy(
                k_ref.at[pl.ds(c * BC, BC)],
                kbuf.at[pl.ds(c * BC, BC)],
                copy_sems.at[2 * c],
            )
            kc.start()
            vc = pltpu.make_async_copy(
                v_ref.at[pl.ds(c * BC, BC)],
                vbuf.at[pl.ds(c * BC, BC)],
                copy_sems.at[2 * c + 1],
            )
            vc.start()
            kcopies.append(kc)
            vcopies.append(vc)

        barrier_sem = pltpu.get_barrier_semaphore()
        for j in range(1, N_DEV):
            pl.semaphore_signal(
                barrier_sem, inc=1,
                device_id=((my_pos + j) % N_DEV,),
                device_id_type=pl.DeviceIdType.MESH,
            )
        pl.semaphore_wait(barrier_sem, N_DEV - 1)

        for c in range(NC):
            kcopies[c].wait()
            ps = []
            for b in range(c * BC, (c + 1) * BC):
                qbT = qblk_ref[b]
                kb = kbuf[b]
                s = (kb[0:H, :] + qbT[:, 0:1]) * SCALE
                m = jnp.max(s, axis=1, keepdims=True)
                p = jnp.exp(s - m)
                l = jnp.sum(p, axis=1, keepdims=True)
                mine_ref[pl.ds(b * H, H), D:D + 1] = m
                mine_ref[pl.ds(b * H, H), D + 1:D + 2] = l
                ps.append(p)
            vcopies[c].wait()
            for i, b in enumerate(range(c * BC, (c + 1) * BC)):
                vb = vbuf[b]
                of = vb[0:H, :] + ps[i][:, 0:1]
                hh = lax.broadcasted_iota(jnp.int32, (H, HD), 0)
                blk = lax.broadcasted_iota(jnp.int32, (H, HD), 1) // D
                ofm = jnp.where(hh == blk, of, 0.0)
                ob = ofm[:, 0:D]
                for h in range(1, H):
                    ob = ob + ofm[:, h * D:(h + 1) * D]
                mine_ref[pl.ds(b * H, H), 0:D] = ob

        rdmas = []
        for j in range(1, N_DEV):
            slot = N_DEV - 1 - j
            rdma = pltpu.make_async_remote_copy(
                src_ref=mine_ref,
                dst_ref=comm_ref.at[slot],
                send_sem=send_sems.at[j - 1],
                recv_sem=recv_sems.at[slot],
                device_id=((my_pos + j) % N_DEV,),
                device_id_type=pl.DeviceIdType.MESH,
            )
            rdma.start()
            rdmas.append(rdma)
        for rdma in rdmas:
            rdma.wait()

        m_parts = [mine_ref[:, D:D + 1]] + [
            comm_ref[i, :, D:D + 1] for i in range(N_DEV - 1)
        ]
        l_parts = [mine_ref[:, D + 1:D + 2]] + [
            comm_ref[i, :, D + 1:D + 2] for i in range(N_DEV - 1)
        ]
        m_g = m_parts[0]
        for i in range(1, N_DEV):
            m_g = jnp.maximum(m_g, m_parts[i])
        alphas = [jnp.exp(mp - m_g) for mp in m_parts]
        l_g = alphas[0] * l_parts[0]
        for i in range(1, N_DEV):
            l_g = l_g + alphas[i] * l_parts[i]
        o_acc = alphas[0] * mine_ref[:, 0:D]
        for i in range(1, N_DEV):
            o_acc = o_acc + alphas[i] * comm_ref[i - 1, :, 0:D]
        out_ref[:, :] = o_acc / l_g

    out2 = pl.pallas_call(
        body,
        out_shape=jax.ShapeDtypeStruct((BH, D), jnp.float32),
        in_specs=[
            pl.BlockSpec(memory_space=pltpu.VMEM),
            pl.BlockSpec(memory_space=pl.ANY),
            pl.BlockSpec(memory_space=pl.ANY),
        ],
        out_specs=pl.BlockSpec(memory_space=pltpu.VMEM),
        scratch_shapes=[
            pltpu.VMEM((B, Kl, HD), jnp.float32),
            pltpu.VMEM((B, Kl, HD), jnp.float32),
            pltpu.VMEM((BH, CW), jnp.float32),
            pltpu.VMEM((N_DEV - 1, BH, CW), jnp.float32),
            pltpu.SemaphoreType.DMA((2 * NC,)),
            pltpu.SemaphoreType.DMA((N_DEV - 1,)),
            pltpu.SemaphoreType.DMA((N_DEV - 1,)),
        ],
        compiler_params=pltpu.CompilerParams(collective_id=0),
    )(qblk, K2, V2)
    return out2.reshape(B, 1, H, D)


# device time: 21623 ns/iter; 1.0816x vs baseline; 1.0217x over previous
import jax
import jax.numpy as jnp
from jax import lax
from jax.experimental import pallas as pl
from jax.experimental.pallas import tpu as pltpu

N_DEV = 4
B, H, D = 8, 8, 64
BH = B * H
HD = H * D
SCALE = D ** -0.5
BC = 2
NC = B // BC
CW = 128


def kernel(Q, K, V):
    Kl = K.shape[1]
    q = Q[:, 0]
    eye8 = jnp.eye(H, dtype=Q.dtype)
    qblk = (q[:, :, None, :] * eye8[None, :, :, None]).reshape(B, H, HD)
    K2 = K.reshape(B, Kl, HD)
    V2 = V.reshape(B, Kl, HD)

    def body(qblk_ref, k_ref, v_ref, out_ref,
             mine_ref, comm_ref, send_sems, recv_sems):
        my_pos = lax.axis_index("i")

        barrier_sem = pltpu.get_barrier_semaphore()
        for j in range(1, N_DEV):
            pl.semaphore_signal(
                barrier_sem, inc=1,
                device_id=((my_pos + j) % N_DEV,),
                device_id_type=pl.DeviceIdType.MESH,
            )
        pl.semaphore_wait(barrier_sem, N_DEV - 1)

        for c in range(NC):
            ps = []
            for b in range(c * BC, (c + 1) * BC):
                qbT = qblk_ref[b]
                kb = k_ref[b]
                s = lax.dot_general(
                    qbT, kb,
                    dimension_numbers=(((1,), (1,)), ((), ())),
                    preferred_element_type=jnp.float32,
                ) * SCALE
                m = jnp.max(s, axis=1, keepdims=True)
                p = jnp.exp(s - m)
                l = jnp.sum(p, axis=1, keepdims=True)
                mine_ref[pl.ds(b * H, H), D:D + 1] = m
                mine_ref[pl.ds(b * H, H), D + 1:D + 2] = l
                ps.append(p)
            for i, b in enumerate(range(c * BC, (c + 1) * BC)):
                vb = v_ref[b]
                of = lax.dot_general(
                    ps[i], vb,
                    dimension_numbers=(((1,), (0,)), ((), ())),
                    preferred_element_type=jnp.float32,
                )
                hh = lax.broadcasted_iota(jnp.int32, (H, HD), 0)
                blk = lax.broadcasted_iota(jnp.int32, (H, HD), 1) // D
                ofm = jnp.where(hh == blk, of, 0.0)
                ob = ofm[:, 0:D]
                for h in range(1, H):
                    ob = ob + ofm[:, h * D:(h + 1) * D]
                mine_ref[pl.ds(b * H, H), 0:D] = ob

        rdmas = []
        for j in range(1, N_DEV):
            slot = N_DEV - 1 - j
            rdma = pltpu.make_async_remote_copy(
                src_ref=mine_ref,
                dst_ref=comm_ref.at[slot],
                send_sem=send_sems.at[j - 1],
                recv_sem=recv_sems.at[slot],
                device_id=((my_pos + j) % N_DEV,),
                device_id_type=pl.DeviceIdType.MESH,
            )
            rdma.start()
            rdmas.append(rdma)
        for rdma in rdmas:
            rdma.wait()

        m_parts = [mine_ref[:, D:D + 1]] + [
            comm_ref[i, :, D:D + 1] for i in range(N_DEV - 1)
        ]
        l_parts = [mine_ref[:, D + 1:D + 2]] + [
            comm_ref[i, :, D + 1:D + 2] for i in range(N_DEV - 1)
        ]
        m_g = m_parts[0]
        for i in range(1, N_DEV):
            m_g = jnp.maximum(m_g, m_parts[i])
        alphas = [jnp.exp(mp - m_g) for mp in m_parts]
        l_g = alphas[0] * l_parts[0]
        for i in range(1, N_DEV):
            l_g = l_g + alphas[i] * l_parts[i]
        o_acc = alphas[0] * mine_ref[:, 0:D]
        for i in range(1, N_DEV):
            o_acc = o_acc + alphas[i] * comm_ref[i - 1, :, 0:D]
        out_ref[:, :] = o_acc / l_g

    out2 = pl.pallas_call(
        body,
        out_shape=jax.ShapeDtypeStruct((BH, D), jnp.float32),
        in_specs=[
            pl.BlockSpec(memory_space=pltpu.VMEM),
            pl.BlockSpec(memory_space=pltpu.VMEM),
            pl.BlockSpec(memory_space=pltpu.VMEM),
        ],
        out_specs=pl.BlockSpec(memory_space=pltpu.VMEM),
        scratch_shapes=[
            pltpu.VMEM((BH, CW), jnp.float32),
            pltpu.VMEM((N_DEV - 1, BH, CW), jnp.float32),
            pltpu.SemaphoreType.DMA((N_DEV - 1,)),
            pltpu.SemaphoreType.DMA((N_DEV - 1,)),
        ],
        compiler_params=pltpu.CompilerParams(collective_id=0),
    )(qblk, K2, V2)
    return out2.reshape(B, 1, H, D)
